# baseline (speedup 1.0000x reference)
import functools

import jax
import jax.numpy as jnp
from jax import lax
from jax.experimental import pallas as pl
from jax.experimental.pallas import tpu as pltpu
from jax.experimental.pallas import tpu_sc as plsc

D = 1000
DP = 1024
B_TOTAL = 20480
NW = 32
B_PER_W = B_TOTAL // NW
CHUNK = 40
NCHUNK = B_PER_W // CHUNK


def _sc_gather(table, idx):
    mesh = plsc.VectorSubcoreMesh(core_axis_name="c", subcore_axis_name="s")

    @functools.partial(
        pl.kernel,
        mesh=mesh,
        out_type=jax.ShapeDtypeStruct((B_TOTAL, DP), jnp.float32),
        scratch_types=[
            pltpu.VMEM((B_PER_W,), jnp.int32),
            pltpu.VMEM((2, CHUNK, DP), jnp.float32),
            pltpu.SemaphoreType.DMA,
            pltpu.SemaphoreType.DMA,
            pltpu.SemaphoreType.DMA,
            pltpu.SemaphoreType.DMA,
        ],
    )
    def k(table_hbm, idx_hbm, out_hbm, idx_v, rows_v, g0, g1, s0, s1):
        wid = lax.axis_index("s") * 2 + lax.axis_index("c")
        base = wid * B_PER_W
        gsem = (g0, g1)
        ssem = (s0, s1)
        pltpu.sync_copy(idx_hbm.at[pl.ds(base, B_PER_W)], idx_v)

        def gather(c, b):
            return pltpu.async_copy(
                table_hbm.at[idx_v.at[pl.ds(c * CHUNK, CHUNK)]],
                rows_v.at[b],
                gsem[b],
            )

        gathers = [gather(0, 0), None]
        scatters = [None, None]
        for c in range(NCHUNK):
            b = c % 2
            gathers[b].wait()
            if c + 1 < NCHUNK:
                nb = (c + 1) % 2
                if scatters[nb] is not None:
                    scatters[nb].wait()
                gathers[nb] = gather(c + 1, nb)
            scatters[b] = pltpu.async_copy(
                rows_v.at[b],
                out_hbm.at[pl.ds(base + c * CHUNK, CHUNK)],
                ssem[b],
            )
        scatters[0].wait()
        scatters[1].wait()

    return k(table, idx)


def kernel(x, y, table):
    idx = x.reshape(-1).astype(jnp.int32)
    table_p = jnp.pad(table, ((0, 0), (0, DP - D)))
    out = _sc_gather(table_p, idx)
    return out[:, :D].reshape(x.shape[0], x.shape[1], D)


# speedup vs baseline: 1.0046x; 1.0046x over previous
import functools

import jax
import jax.numpy as jnp
from jax import lax
from jax.experimental import pallas as pl
from jax.experimental.pallas import tpu as pltpu
from jax.experimental.pallas import tpu_sc as plsc

D = 1000
V = 1000
BATCH = 1024
L = 20
B_TOTAL = BATCH * L
NW = 32
B_PER_W = B_TOTAL // NW
CHUNK = 40
NCHUNK = B_PER_W // CHUNK
BATCH_PER_W = BATCH // NW


def _sc_gather(table, idx):
    mesh = plsc.VectorSubcoreMesh(core_axis_name="c", subcore_axis_name="s")

    @functools.partial(
        pl.kernel,
        mesh=mesh,
        compiler_params=pltpu.CompilerParams(use_tc_tiling_on_sc=False),
        out_type=jax.ShapeDtypeStruct((BATCH, L, D), jnp.float32),
        scratch_types=[
            pltpu.VMEM((B_PER_W,), jnp.int32),
            pltpu.VMEM((2, CHUNK, D), jnp.float32),
            pltpu.SemaphoreType.DMA,
            pltpu.SemaphoreType.DMA,
            pltpu.SemaphoreType.DMA,
            pltpu.SemaphoreType.DMA,
        ],
    )
    def k(table_hbm, idx_hbm, out_hbm, idx_v, rows_v, g0, g1, s0, s1):
        sid = lax.axis_index("s")
        wid = sid * 2 + lax.axis_index("c")
        base = wid * B_PER_W
        bbase = wid * BATCH_PER_W
        gsem = (g0, g1)
        ssem = (s0, s1)
        pltpu.sync_copy(idx_hbm.at[pl.ds(base, B_PER_W)], idx_v)

        def gather(c, b):
            return pltpu.async_copy(
                table_hbm.at[idx_v.at[pl.ds(c * CHUNK, CHUNK)]],
                rows_v.at[b],
                gsem[b],
            )

        def scatter(c, b):
            first = bbase + 2 * c
            cp0 = pltpu.async_copy(
                rows_v.at[b, pl.ds(0, L)], out_hbm.at[first], ssem[b]
            )
            cp1 = pltpu.async_copy(
                rows_v.at[b, pl.ds(L, L)], out_hbm.at[first + 1], ssem[b]
            )
            return (cp0, cp1)

        gathers = [gather(0, 0), None]
        scatters = [None, None]
        for c in range(NCHUNK):
            b = c % 2
            gathers[b].wait()
            if c + 1 < NCHUNK:
                nb = (c + 1) % 2
                if scatters[nb] is not None:
                    scatters[nb][0].wait()
                    scatters[nb][1].wait()
                gathers[nb] = gather(c + 1, nb)
            scatters[b] = scatter(c, b)
        scatters[0][0].wait()
        scatters[0][1].wait()
        scatters[1][0].wait()
        scatters[1][1].wait()

    return k(table, idx)


def kernel(x, y, table):
    idx = x.reshape(-1).astype(jnp.int32)
    return _sc_gather(table, idx)
